# baseline (speedup 1.0000x reference)
import functools

import jax
import jax.numpy as jnp
from jax import lax
from jax.experimental import pallas as pl
from jax.experimental.pallas import tpu as pltpu
from jax.experimental.pallas import tpu_sc as plsc

VOCAB = 1_000_000
DIM_E = 64
BATCH = 4096
HIST = 200

_NC = 2
_NS = 16
_NW = _NC * _NS

_WIDE = VOCAB // 2 + 32
_FULL_COLS = VOCAB // 128
_COLS_BASE = _FULL_COLS // _NW
_COLS_EXTRA = _FULL_COLS % _NW
_BPW = BATCH // _NW


@functools.partial(
    pl.kernel,
    mesh=plsc.VectorSubcoreMesh(core_axis_name="c", subcore_axis_name="s"),
    out_type=jax.ShapeDtypeStruct((_WIDE, 128), jnp.float32),
    scratch_types=[
        pltpu.VMEM((64, 128), jnp.float32),
        pltpu.VMEM((64, 128), jnp.float32),
    ],
    compiler_params=pltpu.CompilerParams(use_tc_tiling_on_sc=True, needs_layout_passes=False),
)
def _transpose_kernel(tt_hbm, tail_hbm, twide_hbm, ibuf, obuf):
    wid = lax.axis_index("s") * _NC + lax.axis_index("c")
    ncols = _COLS_BASE + jnp.where(wid < _COLS_EXTRA, 1, 0)
    base = wid * _COLS_BASE + lax.min(wid, _COLS_EXTRA)
    it = lax.iota(jnp.int32, 16)
    rows_k = [it + 16 * k for k in range(4)]

    def transpose_block(n_wide):
        for w in range(n_wide):
            for p in range(2):
                cols = jnp.full((16,), 2 * w + p, jnp.int32)
                for k in range(4):
                    v = plsc.load_gather(ibuf, [rows_k[k], cols])
                    obuf[w, pl.ds(p * 64 + 16 * k, 16)] = v

    def body(ci, carry):
        c = base + ci
        pltpu.sync_copy(tt_hbm.at[pl.ds(0, 64), pl.ds(c * 128, 128)], ibuf)
        transpose_block(64)
        pltpu.sync_copy(obuf, twide_hbm.at[pl.ds(c * 64, 64)])
        return carry

    lax.fori_loop(0, ncols, body, 0)

    @pl.when(wid == _NW - 1)
    def _():
        pltpu.sync_copy(tail_hbm, ibuf)
        transpose_block(64)
        pltpu.sync_copy(obuf, twide_hbm.at[pl.ds(_FULL_COLS * 64, 64)])


@functools.partial(
    pl.kernel,
    mesh=plsc.VectorSubcoreMesh(core_axis_name="c", subcore_axis_name="s"),
    out_type=jax.ShapeDtypeStruct((HIST, DIM_E, BATCH), jnp.float32),
    scratch_types=[
        pltpu.VMEM((HIST, _BPW), jnp.int32),
        pltpu.VMEM((HIST, _BPW), jnp.int32),
        pltpu.VMEM((_BPW, 128), jnp.float32),
        pltpu.VMEM((DIM_E, _BPW), jnp.float32),
        pltpu.SemaphoreType.DMA,
    ],
    compiler_params=pltpu.CompilerParams(use_tc_tiling_on_sc=True, needs_layout_passes=False),
)
def _gather_kernel(widx_hbm, par_hbm, twide_hbm, out_hbm,
                   widx_v, par_v, wide_v, blk_v, gsem):
    wid = lax.axis_index("s") * _NC + lax.axis_index("c")
    pltpu.sync_copy(widx_hbm.at[wid], widx_v)
    pltpu.sync_copy(par_hbm.at[wid], par_v)
    it = lax.iota(jnp.int32, 16)
    rows_j = [it + 16 * j for j in range(8)]

    def body(h, carry):
        pltpu.async_copy(twide_hbm.at[widx_v.at[h]], wide_v, gsem).wait()
        colbase = [par_v[h, pl.ds(16 * j, 16)] * 64 for j in range(8)]
        for e in range(DIM_E):
            for j in range(8):
                v = plsc.load_gather(wide_v, [rows_j[j], colbase[j] + e])
                blk_v[e, pl.ds(16 * j, 16)] = v
        pltpu.sync_copy(
            blk_v, out_hbm.at[h, pl.ds(0, DIM_E), pl.ds(wid * _BPW, _BPW)])
        return carry

    lax.fori_loop(0, HIST, body, 0)


def kernel(y, table):
    tt = table.T
    tail = jnp.pad(tt[:, _FULL_COLS * 128:], ((0, 0), (0, 64)))
    yb = y.astype(jnp.int32).reshape(_NW, _BPW, HIST)
    widx = (yb >> 1).transpose(0, 2, 1)
    par = (yb & 1).transpose(0, 2, 1)
    twide = _transpose_kernel(tt, tail)
    o_t = _gather_kernel(widx, par, twide)
    return o_t.transpose(2, 0, 1)


# speedup vs baseline: 2.8403x; 2.8403x over previous
import functools

import jax
import jax.numpy as jnp
from jax import lax
from jax.experimental import pallas as pl
from jax.experimental.pallas import tpu as pltpu
from jax.experimental.pallas import tpu_sc as plsc

VOCAB = 1_000_000
DIM_E = 64
BATCH = 4096
HIST = 200

_NC = 2
_NS = 16
_NW = _NC * _NS

_PER_W = BATCH // _NW
_NBUF = 4
_LA = 2
_NGRP = _PER_W // _NBUF


@functools.partial(
    pl.kernel,
    mesh=plsc.VectorSubcoreMesh(core_axis_name="c", subcore_axis_name="s"),
    out_type=jax.ShapeDtypeStruct((BATCH, HIST, DIM_E), jnp.float32),
    scratch_types=[
        pltpu.VMEM((_PER_W, HIST), jnp.int32),
        pltpu.VMEM((_NBUF, HIST, DIM_E), jnp.float32),
        pltpu.SemaphoreType.DMA((_NBUF,)),
        pltpu.SemaphoreType.DMA((_NBUF,)),
    ],
    compiler_params=pltpu.CompilerParams(use_tc_tiling_on_sc=False),
)
def _gather_kernel(idx_hbm, table_hbm, out_hbm, idx_v, rows_v, gsem, osem):
    wid = lax.axis_index("s") * _NC + lax.axis_index("c")
    base = wid * _PER_W
    pltpu.sync_copy(idx_hbm.at[wid], idx_v)

    def gather(j, b):
        return pltpu.make_async_copy(
            table_hbm.at[idx_v.at[j]], rows_v.at[b], gsem.at[b])

    def out_copy(j, b):
        return pltpu.make_async_copy(
            rows_v.at[b], out_hbm.at[base + j], osem.at[b])

    for k in range(_NBUF):
        gather(k, k).start()
        jc = k - _LA
        if jc >= 0:
            bc = jc % _NBUF
            gather(jc, bc).wait()
            out_copy(jc, bc).start()

    def body(g, carry):
        for k in range(_NBUF):
            j = g * _NBUF + k
            out_copy(j - _NBUF, k).wait()
            gather(j, k).start()
            jc = j - _LA
            bc = (k - _LA) % _NBUF
            gather(jc, bc).wait()
            out_copy(jc, bc).start()
        return carry

    lax.fori_loop(1, _NGRP, body, 0)

    for jc in range(_PER_W - _LA, _PER_W):
        bc = jc % _NBUF
        gather(jc, bc).wait()
        out_copy(jc, bc).start()
    for k in range(_NBUF):
        out_copy(_PER_W - _NBUF + k, k).wait()


def kernel(y, table):
    idx = y.astype(jnp.int32).reshape(_NW, _PER_W, HIST)
    return _gather_kernel(idx, table)
